# baseline (speedup 1.0000x reference)
import functools

import jax
import jax.numpy as jnp
from jax import lax
from jax.experimental import pallas as pl
from jax.experimental.pallas import tpu as pltpu
from jax.experimental.pallas import tpu_sc as plsc

N_NODES = 10000
N_EDGES = 320000
DIM = 128
HALF = DIM // 2
NUM_GRAPHS = 64
BN_EPS = 1e-5

NC = 2
NS = 16
LANES = 16
CHUNK = 128
CHUNKS_PER_TILE = 160
EDGES_PER_TILE = CHUNK * CHUNKS_PER_TILE
E_PAD = EDGES_PER_TILE * NS
NPAD = 10240
ROWS_PER_SUB = NPAD // NS
TROWS_PER_SUB = N_NODES // NS
NBUF = 8
NBLK = 4
BLK_CHUNKS = CHUNKS_PER_TILE // NBLK


def _sc_segsum_body(h_hbm, pk_hbm, zeros_hbm, out_hbm,
                    acc_sh, pidx_v, sidx_v, didx_v, rows_v,
                    gsems, ssems, psems):
    c = lax.axis_index("c")
    s = lax.axis_index("s")
    row0 = s * ROWS_PER_SUB
    trow0 = s * TROWS_PER_SUB

    def stage(qb, half):
        return pltpu.make_async_copy(pk_hbm.at[s, qb], pidx_v.at[half],
                                     psems[half])

    zcp = pltpu.make_async_copy(zeros_hbm.at[pl.ds(row0, ROWS_PER_SUB)],
                                acc_sh.at[pl.ds(row0, ROWS_PER_SUB)],
                                gsems[0])
    zcp.start()
    stage(0, 0).start()
    stage(1, 1).start()
    zcp.wait()
    plsc.subcore_barrier()

    mask16 = jnp.full((LANES,), 0xFFFF, jnp.int32)
    hbm_off = jnp.broadcast_to(c * N_NODES, (LANES,))

    def unpack(half, ch, b):
        for k in range(CHUNK // LANES):
            p = pidx_v[half, ch, pl.ds(k * LANES, LANES)]
            sl = (p & mask16) + hbm_off
            sidx_v[b, pl.ds(k * LANES, LANES)] = sl
            didx_v[b, pl.ds(k * LANES, LANES)] = lax.shift_right_logical(p, 16)

    def gather(b):
        return pltpu.make_async_copy(h_hbm.at[sidx_v.at[b]], rows_v[b],
                                     gsems[b])

    for qb in range(NBLK):
        half = qb % 2
        stage(qb, half).wait()

        for b in range(NBUF):
            unpack(half, b, b)
            gather(b).start()

        def body(j, carry):
            ch0 = j * NBUF
            for b in range(NBUF):
                gather(b).wait()
                pltpu.async_copy(rows_v[b], acc_sh.at[didx_v.at[b]],
                                 ssems[b], add=True)
            for b in range(NBUF):
                ch = ch0 + b
                pltpu.make_async_copy(rows_v[b], acc_sh.at[didx_v.at[b]],
                                      ssems[b]).wait()

                @pl.when(ch + NBUF < BLK_CHUNKS)
                def _():
                    unpack(half, ch + NBUF, b)
                    gather(b).start()
            return carry

        lax.fori_loop(0, BLK_CHUNKS // NBUF, body, 0)
        if qb + 2 < NBLK:
            stage(qb + 2, half).start()

    plsc.subcore_barrier()
    pltpu.sync_copy(acc_sh.at[pl.ds(row0, ROWS_PER_SUB)],
                    out_hbm.at[c, pl.ds(row0, ROWS_PER_SUB)])


@functools.cache
def _make_sc_segsum():
    mesh = plsc.VectorSubcoreMesh(core_axis_name="c", subcore_axis_name="s")
    return pl.kernel(
        _sc_segsum_body,
        out_type=jax.ShapeDtypeStruct((NC, NPAD, HALF), jnp.float32),
        mesh=mesh,
        compiler_params=pltpu.CompilerParams(use_tc_tiling_on_sc=False),
        scratch_types=[
            pltpu.VMEM_SHARED((NPAD, HALF), jnp.float32),
            pltpu.VMEM((2, BLK_CHUNKS, CHUNK), jnp.int32),
            pltpu.VMEM((NBUF, CHUNK), jnp.int32),
            pltpu.VMEM((NBUF, CHUNK), jnp.int32),
            [pltpu.VMEM((CHUNK, HALF), jnp.float32) for _ in range(NBUF)],
            [pltpu.SemaphoreType.DMA for _ in range(NBUF)],
            [pltpu.SemaphoreType.DMA for _ in range(NBUF)],
            [pltpu.SemaphoreType.DMA for _ in range(2)],
        ],
    )


def _cat_to_z(hcat_ref, agg_ref):
    z_lo = hcat_ref[:N_NODES, :] + agg_ref[0, :N_NODES, :]
    z_hi = hcat_ref[N_NODES:, :] + agg_ref[1, :N_NODES, :]
    return jnp.concatenate([z_lo, z_hi], axis=1)


def _tc_layer_body(hcat_ref, agg_ref, wa_ref, ba_ref, wb_ref, bb_ref,
                   g_ref, be_ref, out_ref):
    z = _cat_to_z(hcat_ref, agg_ref)
    t = jnp.dot(z, wa_ref[...], preferred_element_type=jnp.float32) + ba_ref[...]
    t = jnp.maximum(t, 0.0)
    u = jnp.dot(t, wb_ref[...], preferred_element_type=jnp.float32) + bb_ref[...]
    mean = jnp.mean(u, axis=0, keepdims=True)
    var = jnp.mean(u * u, axis=0, keepdims=True) - mean * mean
    un = (u - mean) * lax.rsqrt(var + BN_EPS) * g_ref[...] + be_ref[...]
    un = jnp.maximum(un, 0.0)
    out_ref[:N_NODES, :] = un[:, :HALF]
    out_ref[N_NODES:, :] = un[:, HALF:]


def _tc_final_body(hcat_ref, agg_ref, wa_ref, ba_ref, wbr_ref, bb_ref,
                   g_ref, be_ref, batch_ref, out_ref):
    z = _cat_to_z(hcat_ref, agg_ref)
    t = jnp.dot(z, wa_ref[...], preferred_element_type=jnp.float32) + ba_ref[...]
    t = jnp.maximum(t, 0.0)
    u = jnp.sum(t * wbr_ref[...], axis=1, keepdims=True) + bb_ref[...]
    mean = jnp.mean(u, axis=0, keepdims=True)
    var = jnp.mean(u * u, axis=0, keepdims=True) - mean * mean
    v = (u - mean) * lax.rsqrt(var + BN_EPS) * g_ref[...] + be_ref[...]
    xs = v / 5.0
    gids = lax.broadcasted_iota(jnp.int32, (1, NUM_GRAPHS), 1)
    mask = batch_ref[...] == gids
    neg = jnp.float32(-jnp.inf)
    m = jnp.max(jnp.where(mask, xs, neg), axis=0, keepdims=True)
    mrow = jnp.sum(jnp.where(mask, m, 0.0), axis=1, keepdims=True)
    e = jnp.exp(xs - mrow)
    ssum = jnp.sum(jnp.where(mask, e, 0.0), axis=0, keepdims=True)
    srow = jnp.sum(jnp.where(mask, ssum, 0.0), axis=1, keepdims=True)
    out_ref[...] = e / srow


_tc_layer = pl.pallas_call(
    _tc_layer_body,
    out_shape=jax.ShapeDtypeStruct((2 * N_NODES, HALF), jnp.float32),
)

_tc_final = pl.pallas_call(
    _tc_final_body,
    out_shape=jax.ShapeDtypeStruct((N_NODES, 1), jnp.float32),
)


def kernel(x, edge_index, edge_attr, batch,
           W0a, b0a, W0b, b0b, g0, be0,
           W1a, b1a, W1b, b1b, g1, be1,
           W2a, b2a, W2b, b2b, g2, be2):
    del edge_attr
    src = edge_index[0].astype(jnp.int32)
    dst = edge_index[1].astype(jnp.int32)
    pk = src | (dst << 16)
    padv = jnp.full((E_PAD - N_EDGES,), (NPAD - 1) << 16, jnp.int32)
    pk_r = jnp.concatenate([pk, padv]).reshape(NS, NBLK, BLK_CHUNKS, CHUNK)
    zeros_hbm = jnp.zeros((NPAD, HALF), jnp.float32)
    batch2 = batch.astype(jnp.int32).reshape(N_NODES, 1)
    x_cat = jnp.concatenate([x[:, :HALF], x[:, HALF:]], axis=0)

    def r2(v):
        return v.reshape(1, -1)

    _sc_segsum = _make_sc_segsum()
    agg = _sc_segsum(x_cat, pk_r, zeros_hbm)
    h = _tc_layer(x_cat, agg, W0a, r2(b0a), W0b, r2(b0b), r2(g0), r2(be0))
    agg = _sc_segsum(h, pk_r, zeros_hbm)
    h = _tc_layer(h, agg, W1a, r2(b1a), W1b, r2(b1b), r2(g1), r2(be1))
    agg = _sc_segsum(h, pk_r, zeros_hbm)
    out = _tc_final(h, agg, W2a, r2(b2a), W2b.reshape(1, DIM), r2(b2b),
                    r2(g2), r2(be2), batch2)
    return out


# speedup vs baseline: 1.1253x; 1.1253x over previous
import functools

import jax
import jax.numpy as jnp
from jax import lax
from jax.experimental import pallas as pl
from jax.experimental.pallas import tpu as pltpu
from jax.experimental.pallas import tpu_sc as plsc

N_NODES = 10000
N_EDGES = 320000
DIM = 128
HALF = DIM // 2
NUM_GRAPHS = 64
BN_EPS = 1e-5

NC = 2
NS = 16
LANES = 16
CHUNK = 128
CHUNKS_PER_TILE = 160
EDGES_PER_TILE = CHUNK * CHUNKS_PER_TILE
E_PAD = EDGES_PER_TILE * NS
NPAD = 10240
ROWS_PER_SUB = NPAD // NS
TROWS_PER_SUB = N_NODES // NS
NBUF = 4
NBLK = 4
BLK_CHUNKS = CHUNKS_PER_TILE // NBLK


def _sc_segsum_body(h_hbm, pk_hbm, zeros_hbm, out_hbm,
                    acc_sh, tab_sh, pidx_v, sidx_v, didx_v, rows_v,
                    gsems, ssems, psems):
    c = lax.axis_index("c")
    s = lax.axis_index("s")
    row0 = s * ROWS_PER_SUB
    trow0 = s * TROWS_PER_SUB

    def stage(qb, half):
        return pltpu.make_async_copy(pk_hbm.at[s, qb], pidx_v.at[half],
                                     psems[half])

    zcp = pltpu.make_async_copy(zeros_hbm.at[pl.ds(row0, ROWS_PER_SUB)],
                                acc_sh.at[pl.ds(row0, ROWS_PER_SUB)],
                                gsems[0])
    tcp = pltpu.make_async_copy(
        h_hbm.at[pl.ds(c * N_NODES + trow0, TROWS_PER_SUB)],
        tab_sh.at[pl.ds(trow0, TROWS_PER_SUB)], gsems[1])
    zcp.start()
    tcp.start()
    stage(0, 0).start()
    stage(1, 1).start()
    zcp.wait()
    tcp.wait()
    plsc.subcore_barrier()

    mask16 = jnp.full((LANES,), 0xFFFF, jnp.int32)
    hbm_off = jnp.broadcast_to(c * N_NODES, (LANES,))

    def unpack(half, ch, b):
        for k in range(CHUNK // LANES):
            p = pidx_v[half, ch, pl.ds(k * LANES, LANES)]
            sl = p & mask16
            if b != 2:
                sl = sl + hbm_off
            sidx_v[b, pl.ds(k * LANES, LANES)] = sl
            didx_v[b, pl.ds(k * LANES, LANES)] = lax.shift_right_logical(p, 16)

    def gather(b):
        src = h_hbm if b != 2 else tab_sh
        return pltpu.make_async_copy(src.at[sidx_v.at[b]], rows_v[b],
                                     gsems[b])

    for qb in range(NBLK):
        half = qb % 2
        stage(qb, half).wait()

        for b in range(NBUF):
            unpack(half, b, b)
            gather(b).start()

        def body(j, carry):
            ch0 = j * NBUF
            for b in range(NBUF):
                gather(b).wait()
                pltpu.async_copy(rows_v[b], acc_sh.at[didx_v.at[b]],
                                 ssems[b], add=True)
            for b in range(NBUF):
                ch = ch0 + b
                pltpu.make_async_copy(rows_v[b], acc_sh.at[didx_v.at[b]],
                                      ssems[b]).wait()

                @pl.when(ch + NBUF < BLK_CHUNKS)
                def _():
                    unpack(half, ch + NBUF, b)
                    gather(b).start()
            return carry

        lax.fori_loop(0, BLK_CHUNKS // NBUF, body, 0)
        if qb + 2 < NBLK:
            stage(qb + 2, half).start()

    plsc.subcore_barrier()
    pltpu.sync_copy(acc_sh.at[pl.ds(row0, ROWS_PER_SUB)],
                    out_hbm.at[c, pl.ds(row0, ROWS_PER_SUB)])


@functools.cache
def _make_sc_segsum():
    mesh = plsc.VectorSubcoreMesh(core_axis_name="c", subcore_axis_name="s")
    return pl.kernel(
        _sc_segsum_body,
        out_type=jax.ShapeDtypeStruct((NC, NPAD, HALF), jnp.float32),
        mesh=mesh,
        compiler_params=pltpu.CompilerParams(use_tc_tiling_on_sc=False),
        scratch_types=[
            pltpu.VMEM_SHARED((NPAD, HALF), jnp.float32),
            pltpu.VMEM_SHARED((N_NODES, HALF), jnp.float32),
            pltpu.VMEM((2, BLK_CHUNKS, CHUNK), jnp.int32),
            pltpu.VMEM((NBUF, CHUNK), jnp.int32),
            pltpu.VMEM((NBUF, CHUNK), jnp.int32),
            [pltpu.VMEM((CHUNK, HALF), jnp.float32) for _ in range(NBUF)],
            [pltpu.SemaphoreType.DMA for _ in range(NBUF)],
            [pltpu.SemaphoreType.DMA for _ in range(NBUF)],
            [pltpu.SemaphoreType.DMA for _ in range(2)],
        ],
    )


def _cat_to_z(hcat_ref, agg_ref):
    z_lo = hcat_ref[:N_NODES, :] + agg_ref[0, :N_NODES, :]
    z_hi = hcat_ref[N_NODES:, :] + agg_ref[1, :N_NODES, :]
    return jnp.concatenate([z_lo, z_hi], axis=1)


def _tc_layer_body(hcat_ref, agg_ref, wa_ref, ba_ref, wb_ref, bb_ref,
                   g_ref, be_ref, out_ref):
    z = _cat_to_z(hcat_ref, agg_ref)
    t = jnp.dot(z, wa_ref[...], preferred_element_type=jnp.float32) + ba_ref[...]
    t = jnp.maximum(t, 0.0)
    u = jnp.dot(t, wb_ref[...], preferred_element_type=jnp.float32) + bb_ref[...]
    mean = jnp.mean(u, axis=0, keepdims=True)
    var = jnp.mean(u * u, axis=0, keepdims=True) - mean * mean
    un = (u - mean) * lax.rsqrt(var + BN_EPS) * g_ref[...] + be_ref[...]
    un = jnp.maximum(un, 0.0)
    out_ref[:N_NODES, :] = un[:, :HALF]
    out_ref[N_NODES:, :] = un[:, HALF:]


def _tc_final_body(hcat_ref, agg_ref, wa_ref, ba_ref, wbr_ref, bb_ref,
                   g_ref, be_ref, batch_ref, out_ref):
    z = _cat_to_z(hcat_ref, agg_ref)
    t = jnp.dot(z, wa_ref[...], preferred_element_type=jnp.float32) + ba_ref[...]
    t = jnp.maximum(t, 0.0)
    u = jnp.sum(t * wbr_ref[...], axis=1, keepdims=True) + bb_ref[...]
    mean = jnp.mean(u, axis=0, keepdims=True)
    var = jnp.mean(u * u, axis=0, keepdims=True) - mean * mean
    v = (u - mean) * lax.rsqrt(var + BN_EPS) * g_ref[...] + be_ref[...]
    xs = v / 5.0
    gids = lax.broadcasted_iota(jnp.int32, (1, NUM_GRAPHS), 1)
    mask = batch_ref[...] == gids
    neg = jnp.float32(-jnp.inf)
    m = jnp.max(jnp.where(mask, xs, neg), axis=0, keepdims=True)
    mrow = jnp.sum(jnp.where(mask, m, 0.0), axis=1, keepdims=True)
    e = jnp.exp(xs - mrow)
    ssum = jnp.sum(jnp.where(mask, e, 0.0), axis=0, keepdims=True)
    srow = jnp.sum(jnp.where(mask, ssum, 0.0), axis=1, keepdims=True)
    out_ref[...] = e / srow


_tc_layer = pl.pallas_call(
    _tc_layer_body,
    out_shape=jax.ShapeDtypeStruct((2 * N_NODES, HALF), jnp.float32),
)

_tc_final = pl.pallas_call(
    _tc_final_body,
    out_shape=jax.ShapeDtypeStruct((N_NODES, 1), jnp.float32),
)


def kernel(x, edge_index, edge_attr, batch,
           W0a, b0a, W0b, b0b, g0, be0,
           W1a, b1a, W1b, b1b, g1, be1,
           W2a, b2a, W2b, b2b, g2, be2):
    del edge_attr
    src = edge_index[0].astype(jnp.int32)
    dst = edge_index[1].astype(jnp.int32)
    pk = src | (dst << 16)
    padv = jnp.full((E_PAD - N_EDGES,), (NPAD - 1) << 16, jnp.int32)
    pk_r = jnp.concatenate([pk, padv]).reshape(NS, NBLK, BLK_CHUNKS, CHUNK)
    zeros_hbm = jnp.zeros((NPAD, HALF), jnp.float32)
    batch2 = batch.astype(jnp.int32).reshape(N_NODES, 1)
    x_cat = jnp.concatenate([x[:, :HALF], x[:, HALF:]], axis=0)

    def r2(v):
        return v.reshape(1, -1)

    _sc_segsum = _make_sc_segsum()
    agg = _sc_segsum(x_cat, pk_r, zeros_hbm)
    h = _tc_layer(x_cat, agg, W0a, r2(b0a), W0b, r2(b0b), r2(g0), r2(be0))
    agg = _sc_segsum(h, pk_r, zeros_hbm)
    h = _tc_layer(h, agg, W1a, r2(b1a), W1b, r2(b1b), r2(g1), r2(be1))
    agg = _sc_segsum(h, pk_r, zeros_hbm)
    out = _tc_final(h, agg, W2a, r2(b2a), W2b.reshape(1, DIM), r2(b2b),
                    r2(g2), r2(be2), batch2)
    return out


# speedup vs baseline: 1.3028x; 1.1577x over previous
import functools

import jax
import jax.numpy as jnp
from jax import lax
from jax.experimental import pallas as pl
from jax.experimental.pallas import tpu as pltpu
from jax.experimental.pallas import tpu_sc as plsc

N_NODES = 10000
N_EDGES = 320000
DIM = 128
HALF = DIM // 2
NUM_GRAPHS = 64
BN_EPS = 1e-5

NC = 2
NS = 16
LANES = 16
CHUNK = 128
CHUNKS_PER_TILE = 160
EDGES_PER_TILE = CHUNK * CHUNKS_PER_TILE
E_PAD = EDGES_PER_TILE * NS
NPAD = 10240
ROWS_PER_SUB = NPAD // NS
TROWS_PER_SUB = N_NODES // NS
NBUF = 4
NBLK = 4
BLK_CHUNKS = CHUNKS_PER_TILE // NBLK


def _sc_segsum_body(h_hbm, pk_hbm, zeros_hbm, out_hbm,
                    acc_sh, tab_sh, pidx_v, sidx_v, didx_v, rows_v,
                    gsems, ssems, psems):
    c = lax.axis_index("c")
    s = lax.axis_index("s")
    row0 = s * ROWS_PER_SUB
    trow0 = s * TROWS_PER_SUB

    def stage(qb, half):
        return pltpu.make_async_copy(pk_hbm.at[s, qb], pidx_v.at[half],
                                     psems[half])

    zcp = pltpu.make_async_copy(zeros_hbm.at[pl.ds(row0, ROWS_PER_SUB)],
                                acc_sh.at[pl.ds(row0, ROWS_PER_SUB)],
                                gsems[0])
    tcp = pltpu.make_async_copy(
        h_hbm.at[pl.ds(c * N_NODES + trow0, TROWS_PER_SUB)],
        tab_sh.at[pl.ds(trow0, TROWS_PER_SUB)], gsems[1])
    zcp.start()
    tcp.start()
    stage(0, 0).start()
    stage(1, 1).start()
    zcp.wait()
    tcp.wait()
    plsc.subcore_barrier()

    mask16 = jnp.full((LANES,), 0xFFFF, jnp.int32)
    hbm_off = jnp.broadcast_to(c * N_NODES, (LANES,))

    def unpack(half, ch, b):
        for k in range(CHUNK // LANES):
            p = pidx_v[half, ch, pl.ds(k * LANES, LANES)]
            sl = p & mask16
            if b % 2 == 0:
                sl = sl + hbm_off
            sidx_v[b, pl.ds(k * LANES, LANES)] = sl
            didx_v[b, pl.ds(k * LANES, LANES)] = lax.shift_right_logical(p, 16)

    def gather(b):
        src = h_hbm if b % 2 == 0 else tab_sh
        return pltpu.make_async_copy(src.at[sidx_v.at[b]], rows_v[b],
                                     gsems[b])

    for qb in range(NBLK):
        half = qb % 2
        stage(qb, half).wait()

        for b in range(NBUF):
            unpack(half, b, b)
            gather(b).start()

        def body(j, carry):
            ch0 = j * NBUF
            for b in range(NBUF):
                gather(b).wait()
                pltpu.async_copy(rows_v[b], acc_sh.at[didx_v.at[b]],
                                 ssems[b], add=True)
            for b in range(NBUF):
                ch = ch0 + b
                pltpu.make_async_copy(rows_v[b], acc_sh.at[didx_v.at[b]],
                                      ssems[b]).wait()

                @pl.when(ch + NBUF < BLK_CHUNKS)
                def _():
                    unpack(half, ch + NBUF, b)
                    gather(b).start()
            return carry

        lax.fori_loop(0, BLK_CHUNKS // NBUF, body, 0)
        if qb + 2 < NBLK:
            stage(qb + 2, half).start()

    plsc.subcore_barrier()
    pltpu.sync_copy(acc_sh.at[pl.ds(row0, ROWS_PER_SUB)],
                    out_hbm.at[c, pl.ds(row0, ROWS_PER_SUB)])


@functools.cache
def _make_sc_segsum():
    mesh = plsc.VectorSubcoreMesh(core_axis_name="c", subcore_axis_name="s")
    return pl.kernel(
        _sc_segsum_body,
        out_type=jax.ShapeDtypeStruct((NC, NPAD, HALF), jnp.float32),
        mesh=mesh,
        compiler_params=pltpu.CompilerParams(use_tc_tiling_on_sc=False),
        scratch_types=[
            pltpu.VMEM_SHARED((NPAD, HALF), jnp.float32),
            pltpu.VMEM_SHARED((N_NODES, HALF), jnp.float32),
            pltpu.VMEM((2, BLK_CHUNKS, CHUNK), jnp.int32),
            pltpu.VMEM((NBUF, CHUNK), jnp.int32),
            pltpu.VMEM((NBUF, CHUNK), jnp.int32),
            [pltpu.VMEM((CHUNK, HALF), jnp.float32) for _ in range(NBUF)],
            [pltpu.SemaphoreType.DMA for _ in range(NBUF)],
            [pltpu.SemaphoreType.DMA for _ in range(NBUF)],
            [pltpu.SemaphoreType.DMA for _ in range(2)],
        ],
    )


def _cat_to_z(hcat_ref, agg_ref):
    z_lo = hcat_ref[:N_NODES, :] + agg_ref[0, :N_NODES, :]
    z_hi = hcat_ref[N_NODES:, :] + agg_ref[1, :N_NODES, :]
    return jnp.concatenate([z_lo, z_hi], axis=1)


def _tc_layer_body(hcat_ref, agg_ref, wa_ref, ba_ref, wb_ref, bb_ref,
                   g_ref, be_ref, out_ref):
    z = _cat_to_z(hcat_ref, agg_ref)
    t = jnp.dot(z, wa_ref[...], preferred_element_type=jnp.float32) + ba_ref[...]
    t = jnp.maximum(t, 0.0)
    u = jnp.dot(t, wb_ref[...], preferred_element_type=jnp.float32) + bb_ref[...]
    mean = jnp.mean(u, axis=0, keepdims=True)
    var = jnp.mean(u * u, axis=0, keepdims=True) - mean * mean
    un = (u - mean) * lax.rsqrt(var + BN_EPS) * g_ref[...] + be_ref[...]
    un = jnp.maximum(un, 0.0)
    out_ref[:N_NODES, :] = un[:, :HALF]
    out_ref[N_NODES:, :] = un[:, HALF:]


def _tc_final_body(hcat_ref, agg_ref, wa_ref, ba_ref, wbr_ref, bb_ref,
                   g_ref, be_ref, batch_ref, out_ref):
    z = _cat_to_z(hcat_ref, agg_ref)
    t = jnp.dot(z, wa_ref[...], preferred_element_type=jnp.float32) + ba_ref[...]
    t = jnp.maximum(t, 0.0)
    u = jnp.sum(t * wbr_ref[...], axis=1, keepdims=True) + bb_ref[...]
    mean = jnp.mean(u, axis=0, keepdims=True)
    var = jnp.mean(u * u, axis=0, keepdims=True) - mean * mean
    v = (u - mean) * lax.rsqrt(var + BN_EPS) * g_ref[...] + be_ref[...]
    xs = v / 5.0
    gids = lax.broadcasted_iota(jnp.int32, (1, NUM_GRAPHS), 1)
    mask = batch_ref[...] == gids
    neg = jnp.float32(-jnp.inf)
    m = jnp.max(jnp.where(mask, xs, neg), axis=0, keepdims=True)
    mrow = jnp.sum(jnp.where(mask, m, 0.0), axis=1, keepdims=True)
    e = jnp.exp(xs - mrow)
    ssum = jnp.sum(jnp.where(mask, e, 0.0), axis=0, keepdims=True)
    srow = jnp.sum(jnp.where(mask, ssum, 0.0), axis=1, keepdims=True)
    out_ref[...] = e / srow


_tc_layer = pl.pallas_call(
    _tc_layer_body,
    out_shape=jax.ShapeDtypeStruct((2 * N_NODES, HALF), jnp.float32),
)

_tc_final = pl.pallas_call(
    _tc_final_body,
    out_shape=jax.ShapeDtypeStruct((N_NODES, 1), jnp.float32),
)


def kernel(x, edge_index, edge_attr, batch,
           W0a, b0a, W0b, b0b, g0, be0,
           W1a, b1a, W1b, b1b, g1, be1,
           W2a, b2a, W2b, b2b, g2, be2):
    del edge_attr
    src = edge_index[0].astype(jnp.int32)
    dst = edge_index[1].astype(jnp.int32)
    pk = src | (dst << 16)
    padv = jnp.full((E_PAD - N_EDGES,), (NPAD - 1) << 16, jnp.int32)
    pk_r = jnp.concatenate([pk, padv]).reshape(NS, NBLK, BLK_CHUNKS, CHUNK)
    zeros_hbm = jnp.zeros((NPAD, HALF), jnp.float32)
    batch2 = batch.astype(jnp.int32).reshape(N_NODES, 1)
    x_cat = jnp.concatenate([x[:, :HALF], x[:, HALF:]], axis=0)

    def r2(v):
        return v.reshape(1, -1)

    _sc_segsum = _make_sc_segsum()
    agg = _sc_segsum(x_cat, pk_r, zeros_hbm)
    h = _tc_layer(x_cat, agg, W0a, r2(b0a), W0b, r2(b0b), r2(g0), r2(be0))
    agg = _sc_segsum(h, pk_r, zeros_hbm)
    h = _tc_layer(h, agg, W1a, r2(b1a), W1b, r2(b1b), r2(g1), r2(be1))
    agg = _sc_segsum(h, pk_r, zeros_hbm)
    out = _tc_final(h, agg, W2a, r2(b2a), W2b.reshape(1, DIM), r2(b2b),
                    r2(g2), r2(be2), batch2)
    return out
